# baseline (speedup 1.0000x reference)
import functools

import jax
import jax.numpy as jnp
from jax.experimental import pallas as pl

_K = 10
_N = 4096
_D = 128
_BR = 256


def _topk_kernel(x_ref, y_ref, dist_ref, idx_ref):
    x = x_ref[0]
    y = y_ref[0]
    xsq = jnp.sum(x * x, axis=1, keepdims=True)
    ysq = jnp.sum(y * y, axis=1, keepdims=True)
    xy = jax.lax.dot_general(
        x, y, (((1,), (1,)), ((), ())),
        preferred_element_type=jnp.float32,
    )
    s = ysq.T - 2.0 * xy

    q = _N // 4
    base = jax.lax.broadcasted_iota(jnp.int32, (_BR, q), 1).astype(jnp.float32)
    v = [s[:, i * q:(i + 1) * q] for i in range(4)]
    c = [base + float(i * q) for i in range(4)]

    def comp(i, j):
        swap = v[j] < v[i]
        lo = jnp.minimum(v[i], v[j])
        hi = jnp.maximum(v[i], v[j])
        ci = jnp.where(swap, c[j], c[i])
        cj = jnp.where(swap, c[i], c[j])
        v[i], v[j], c[i], c[j] = lo, hi, ci, cj

    for i, j in ((0, 1), (2, 3), (0, 2), (1, 3), (1, 2)):
        comp(i, j)

    vals = []
    idxs = []
    for t in range(_K):
        m = jnp.min(v[0], axis=1, keepdims=True)
        sel = jnp.min(jnp.where(v[0] == m, c[0], jnp.inf), axis=1, keepdims=True)
        vals.append(m)
        idxs.append(sel)
        if t + 1 < _K:
            cond = c[0] == sel
            v[0] = jnp.where(cond, v[1], v[0])
            c[0] = jnp.where(cond, c[1], c[0])
            v[1] = jnp.where(cond, v[2], v[1])
            c[1] = jnp.where(cond, c[2], c[1])
            v[2] = jnp.where(cond, v[3], v[2])
            c[2] = jnp.where(cond, c[3], c[2])
            v[3] = jnp.where(cond, jnp.inf, v[3])
    sv = jnp.concatenate(vals, axis=1)
    dist_ref[0] = jnp.sqrt(jnp.maximum(xsq + sv, 1e-12))
    idx_ref[0] = jnp.concatenate(idxs, axis=1).astype(jnp.int32)


@functools.partial(jax.jit, static_argnames=())
def _graph_topk(ins_feats):
    num_view = ins_feats.shape[0]
    grid = (num_view, _N // _BR)
    out_dist, out_idx = pl.pallas_call(
        _topk_kernel,
        grid=grid,
        in_specs=[
            pl.BlockSpec((1, _BR, _D), lambda v, r: (v, r, 0)),
            pl.BlockSpec((1, _N, _D), lambda v, r: (v, 0, 0)),
        ],
        out_specs=[
            pl.BlockSpec((1, _BR, _K), lambda v, r: (v, r, 0)),
            pl.BlockSpec((1, _BR, _K), lambda v, r: (v, r, 0)),
        ],
        out_shape=[
            jax.ShapeDtypeStruct((num_view, _N, _K), jnp.float32),
            jax.ShapeDtypeStruct((num_view, _N, _K), jnp.int32),
        ],
    )(ins_feats, ins_feats)
    return out_dist, out_idx


def kernel(ins_feats, label_feats, target, k):
    num_view, num_ins, _ = ins_feats.shape
    num_label = label_feats.shape[0]
    k_static = _K
    topk_all, topk_idx = _graph_topk(ins_feats)
    dst_idx = topk_idx.reshape(num_view, -1).astype(jnp.int64)
    src_idx = jnp.repeat(jnp.arange(num_ins, dtype=jnp.int64), k_static)
    src_all = jnp.broadcast_to(src_idx[None, :], dst_idx.shape)
    edge_all = jnp.stack([src_all, dst_idx], axis=1)
    edge_all = edge_all + jnp.asarray(k - k_static, dtype=edge_all.dtype)
    ins_idx_C = jnp.repeat(jnp.arange(num_ins, dtype=jnp.int64), num_view)[:, None]
    view_idx_C = jnp.tile(jnp.arange(num_view, dtype=jnp.int64)[:, None], (num_ins, 1))
    view_edge_index_C = jnp.concatenate([ins_idx_C, view_idx_C], axis=1)
    gins_idx_C = jnp.repeat(jnp.arange(num_ins, dtype=jnp.int64), num_label)[:, None]
    label_idx_C = jnp.tile(jnp.arange(num_label, dtype=jnp.int64)[:, None], (num_ins, 1))
    gedge_index_C = jnp.concatenate([gins_idx_C, label_idx_C], axis=1)
    return (topk_all, edge_all, view_edge_index_C, gedge_index_C, label_feats)


# speedup vs baseline: 1.1101x; 1.1101x over previous
import functools

import jax
import jax.numpy as jnp
from jax.experimental import pallas as pl
from jax.experimental.pallas import tpu as pltpu

_K = 10
_N = 4096
_D = 128
_BR = 512


def _topk_kernel(x_ref, y_ref, dist_ref, idx_ref):
    x = x_ref[0]
    y = y_ref[0]
    xsq = jnp.sum(x * x, axis=1, keepdims=True)
    ysq = jnp.sum(y * y, axis=1, keepdims=True)
    xy = jax.lax.dot_general(
        x, y, (((1,), (1,)), ((), ())),
        preferred_element_type=jnp.float32,
    )
    s = ysq.T - 2.0 * xy

    colf = jax.lax.broadcasted_iota(jnp.int32, (_BR, _N), 1).astype(jnp.float32)
    vals = []
    idxs = []
    d = s
    for t in range(_K):
        m = jnp.min(d, axis=1, keepdims=True)
        sel = jnp.min(jnp.where(d == m, colf, jnp.inf), axis=1, keepdims=True)
        vals.append(m)
        idxs.append(sel)
        if t + 1 < _K:
            d = jnp.where(colf == sel, jnp.inf, d)
    sv = jnp.concatenate(vals, axis=1)
    dist_ref[0] = jnp.sqrt(jnp.maximum(xsq + sv, 1e-12))
    idx_ref[0] = jnp.concatenate(idxs, axis=1).astype(jnp.int32)


@functools.partial(jax.jit, static_argnames=())
def _graph_topk(ins_feats):
    num_view = ins_feats.shape[0]
    grid = (num_view, _N // _BR)
    out_dist, out_idx = pl.pallas_call(
        _topk_kernel,
        grid=grid,
        in_specs=[
            pl.BlockSpec((1, _BR, _D), lambda v, r: (v, r, 0)),
            pl.BlockSpec((1, _N, _D), lambda v, r: (v, 0, 0)),
        ],
        out_specs=[
            pl.BlockSpec((1, _BR, _K), lambda v, r: (v, r, 0)),
            pl.BlockSpec((1, _BR, _K), lambda v, r: (v, r, 0)),
        ],
        out_shape=[
            jax.ShapeDtypeStruct((num_view, _N, _K), jnp.float32),
            jax.ShapeDtypeStruct((num_view, _N, _K), jnp.int32),
        ],
        compiler_params=pltpu.CompilerParams(
            dimension_semantics=("parallel", "parallel")),
    )(ins_feats, ins_feats)
    return out_dist, out_idx


def kernel(ins_feats, label_feats, target, k):
    num_view, num_ins, _ = ins_feats.shape
    num_label = label_feats.shape[0]
    k_static = _K
    topk_all, topk_idx = _graph_topk(ins_feats)
    dst_idx = topk_idx.reshape(num_view, -1).astype(jnp.int64)
    src_idx = jnp.repeat(jnp.arange(num_ins, dtype=jnp.int64), k_static)
    src_all = jnp.broadcast_to(src_idx[None, :], dst_idx.shape)
    edge_all = jnp.stack([src_all, dst_idx], axis=1)
    edge_all = edge_all + jnp.asarray(k - k_static, dtype=edge_all.dtype)
    ins_idx_C = jnp.repeat(jnp.arange(num_ins, dtype=jnp.int64), num_view)[:, None]
    view_idx_C = jnp.tile(jnp.arange(num_view, dtype=jnp.int64)[:, None], (num_ins, 1))
    view_edge_index_C = jnp.concatenate([ins_idx_C, view_idx_C], axis=1)
    gins_idx_C = jnp.repeat(jnp.arange(num_ins, dtype=jnp.int64), num_label)[:, None]
    label_idx_C = jnp.tile(jnp.arange(num_label, dtype=jnp.int64)[:, None], (num_ins, 1))
    gedge_index_C = jnp.concatenate([gins_idx_C, label_idx_C], axis=1)
    return (topk_all, edge_all, view_edge_index_C, gedge_index_C, label_feats)


# speedup vs baseline: 1.1295x; 1.0175x over previous
import functools

import jax
import jax.numpy as jnp
from jax.experimental import pallas as pl
from jax.experimental.pallas import tpu as pltpu

_K = 10
_N = 4096
_D = 128
_BR = 512


def _topk_kernel(x_ref, y_ref, dist_ref, idx_ref):
    x = x_ref[0]
    y = y_ref[0]
    xsq = jnp.sum(x * x, axis=1, keepdims=True)
    ysq = jnp.sum(y * y, axis=1, keepdims=True)
    xy2 = jax.lax.dot_general(
        x, -2.0 * y, (((1,), (1,)), ((), ())),
        preferred_element_type=jnp.float32,
    )
    s = ysq.T + xy2

    colf = jax.lax.broadcasted_iota(jnp.int32, (_BR, _N), 1).astype(jnp.float32)
    vals = []
    idxs = []
    d = s
    for t in range(_K):
        m = jnp.min(d, axis=1, keepdims=True)
        sel = jnp.min(jnp.where(d == m, colf, jnp.inf), axis=1, keepdims=True)
        vals.append(m)
        idxs.append(sel)
        if t + 1 < _K:
            d = jnp.where(colf == sel, jnp.inf, d)
    sv = jnp.concatenate(vals, axis=1)
    dist_ref[0] = jnp.sqrt(jnp.maximum(xsq + sv, 1e-12))
    idx_ref[0] = jnp.concatenate(idxs, axis=1).astype(jnp.int32)


@functools.partial(jax.jit, static_argnames=())
def _graph_topk(ins_feats):
    num_view = ins_feats.shape[0]
    grid = (num_view, _N // _BR)
    out_dist, out_idx = pl.pallas_call(
        _topk_kernel,
        grid=grid,
        in_specs=[
            pl.BlockSpec((1, _BR, _D), lambda v, r: (v, r, 0)),
            pl.BlockSpec((1, _N, _D), lambda v, r: (v, 0, 0)),
        ],
        out_specs=[
            pl.BlockSpec((1, _BR, _K), lambda v, r: (v, r, 0)),
            pl.BlockSpec((1, _BR, _K), lambda v, r: (v, r, 0)),
        ],
        out_shape=[
            jax.ShapeDtypeStruct((num_view, _N, _K), jnp.float32),
            jax.ShapeDtypeStruct((num_view, _N, _K), jnp.int32),
        ],
        compiler_params=pltpu.CompilerParams(
            dimension_semantics=("parallel", "parallel")),
    )(ins_feats, ins_feats)
    return out_dist, out_idx


def kernel(ins_feats, label_feats, target, k):
    num_view, num_ins, _ = ins_feats.shape
    num_label = label_feats.shape[0]
    k_static = _K
    topk_all, topk_idx = _graph_topk(ins_feats)
    dst_idx = topk_idx.reshape(num_view, -1).astype(jnp.int64)
    src_idx = jnp.repeat(jnp.arange(num_ins, dtype=jnp.int64), k_static)
    src_all = jnp.broadcast_to(src_idx[None, :], dst_idx.shape)
    edge_all = jnp.stack([src_all, dst_idx], axis=1)
    edge_all = edge_all + jnp.asarray(k - k_static, dtype=edge_all.dtype)
    ins_idx_C = jnp.repeat(jnp.arange(num_ins, dtype=jnp.int64), num_view)[:, None]
    view_idx_C = jnp.tile(jnp.arange(num_view, dtype=jnp.int64)[:, None], (num_ins, 1))
    view_edge_index_C = jnp.concatenate([ins_idx_C, view_idx_C], axis=1)
    gins_idx_C = jnp.repeat(jnp.arange(num_ins, dtype=jnp.int64), num_label)[:, None]
    label_idx_C = jnp.tile(jnp.arange(num_label, dtype=jnp.int64)[:, None], (num_ins, 1))
    gedge_index_C = jnp.concatenate([gins_idx_C, label_idx_C], axis=1)
    return (topk_all, edge_all, view_edge_index_C, gedge_index_C, label_feats)


# speedup vs baseline: 1.1913x; 1.0547x over previous
import functools

import jax
import jax.numpy as jnp
from jax.experimental import pallas as pl
from jax.experimental.pallas import tpu as pltpu

_K = 10
_N = 4096
_D = 128
_BR = 512


def _topk_kernel(x_ref, y_ref, dist_ref, idx_ref):
    x = x_ref[0]
    y = y_ref[0]
    xsq = jnp.sum(x * x, axis=1, keepdims=True)
    ysq = jnp.sum(y * y, axis=1, keepdims=True)
    x_aug = jnp.concatenate([x, jnp.ones((x.shape[0], 1), jnp.float32)], axis=1)
    y_aug = jnp.concatenate([-2.0 * y, ysq], axis=1)
    s = jax.lax.dot_general(
        x_aug, y_aug, (((1,), (1,)), ((), ())),
        preferred_element_type=jnp.float32,
    )

    colf = jax.lax.broadcasted_iota(jnp.int32, (_BR, _N), 1).astype(jnp.float32)
    vals = []
    idxs = []
    d = s
    for t in range(_K):
        m = jnp.min(d, axis=1, keepdims=True)
        sel = jnp.min(jnp.where(d == m, colf, jnp.inf), axis=1, keepdims=True)
        vals.append(m)
        idxs.append(sel)
        if t + 1 < _K:
            d = jnp.where(colf == sel, jnp.inf, d)
    sv = jnp.concatenate(vals, axis=1)
    dist_ref[0] = jnp.sqrt(jnp.maximum(xsq + sv, 1e-12))
    idx_ref[0] = jnp.concatenate(idxs, axis=1).astype(jnp.int32)


@functools.partial(jax.jit, static_argnames=())
def _graph_topk(ins_feats):
    num_view = ins_feats.shape[0]
    grid = (num_view, _N // _BR)
    out_dist, out_idx = pl.pallas_call(
        _topk_kernel,
        grid=grid,
        in_specs=[
            pl.BlockSpec((1, _BR, _D), lambda v, r: (v, r, 0)),
            pl.BlockSpec((1, _N, _D), lambda v, r: (v, 0, 0)),
        ],
        out_specs=[
            pl.BlockSpec((1, _BR, _K), lambda v, r: (v, r, 0)),
            pl.BlockSpec((1, _BR, _K), lambda v, r: (v, r, 0)),
        ],
        out_shape=[
            jax.ShapeDtypeStruct((num_view, _N, _K), jnp.float32),
            jax.ShapeDtypeStruct((num_view, _N, _K), jnp.int32),
        ],
        compiler_params=pltpu.CompilerParams(
            dimension_semantics=("parallel", "parallel")),
    )(ins_feats, ins_feats)
    return out_dist, out_idx


def kernel(ins_feats, label_feats, target, k):
    num_view, num_ins, _ = ins_feats.shape
    num_label = label_feats.shape[0]
    k_static = _K
    topk_all, topk_idx = _graph_topk(ins_feats)
    dst_idx = topk_idx.reshape(num_view, -1).astype(jnp.int64)
    src_idx = jnp.repeat(jnp.arange(num_ins, dtype=jnp.int64), k_static)
    src_all = jnp.broadcast_to(src_idx[None, :], dst_idx.shape)
    edge_all = jnp.stack([src_all, dst_idx], axis=1)
    edge_all = edge_all + jnp.asarray(k - k_static, dtype=edge_all.dtype)
    ins_idx_C = jnp.repeat(jnp.arange(num_ins, dtype=jnp.int64), num_view)[:, None]
    view_idx_C = jnp.tile(jnp.arange(num_view, dtype=jnp.int64)[:, None], (num_ins, 1))
    view_edge_index_C = jnp.concatenate([ins_idx_C, view_idx_C], axis=1)
    gins_idx_C = jnp.repeat(jnp.arange(num_ins, dtype=jnp.int64), num_label)[:, None]
    label_idx_C = jnp.tile(jnp.arange(num_label, dtype=jnp.int64)[:, None], (num_ins, 1))
    gedge_index_C = jnp.concatenate([gins_idx_C, label_idx_C], axis=1)
    return (topk_all, edge_all, view_edge_index_C, gedge_index_C, label_feats)
